# baseline (speedup 1.0000x reference)
import jax
import jax.numpy as jnp
from jax.experimental import pallas as pl
from jax.experimental.pallas import tpu as pltpu
from jax.experimental.pallas import tpu_sc as plsc

_N = 4096
_BK = 512
_NSLAB = _N // _BK
_SC_SLABS = 3
_SC_ROWS = _SC_SLABS * _BK

_SC_NC = 2
_SC_NS = 16
_SC_NW = _SC_NC * _SC_NS
_SC_ROWS_PER_W = _SC_ROWS // _SC_NW
_SC_CHUNK = 8
_SC_UNROLL = 8


def _sc_deg_body(d_hbm, deg_hbm, buf_ref, degv_ref, sem0, sem1):
    cidx = jax.lax.axis_index("c")
    sidx = jax.lax.axis_index("s")
    wid = sidx * _SC_NC + cidx
    base = wid * _SC_ROWS_PER_W
    nch = _SC_ROWS_PER_W // _SC_CHUNK
    lanes = jax.lax.iota(jnp.int32, 16)
    sems = [sem0, sem1]
    copies = [None, None]
    copies[0] = pltpu.make_async_copy(
        d_hbm.at[pl.ds(base, _SC_CHUNK)], buf_ref.at[0], sem0)
    copies[0].start()
    dv = jnp.zeros((16,), jnp.float32)
    for g in range(nch):
        cur = g % 2
        nxt = (g + 1) % 2
        if g + 1 < nch:
            copies[nxt] = pltpu.make_async_copy(
                d_hbm.at[pl.ds(base + (g + 1) * _SC_CHUNK, _SC_CHUNK)],
                buf_ref.at[nxt], sems[nxt])
            copies[nxt].start()
        copies[cur].wait()
        for r in range(_SC_CHUNK):
            def jbody(j, m, _cur=cur, _r=r):
                for u in range(_SC_UNROLL):
                    m = jnp.maximum(
                        m, buf_ref[_cur, _r, pl.ds(j * (16 * _SC_UNROLL)
                                                   + u * 16, 16)])
                return m
            m = jax.lax.fori_loop(
                0, _N // (16 * _SC_UNROLL), jbody,
                jnp.full((16,), -jnp.inf, jnp.float32))
            for sh in (8, 4, 2, 1):
                m = jnp.maximum(
                    m, m.at[lanes ^ sh].get(mode="promise_in_bounds"))
            dv = jnp.where(lanes == (g % 2) * _SC_CHUNK + r, m, dv)
        if g % 2 == 1:
            degv_ref[pl.ds((g // 2) * 16, 16)] = dv
            dv = jnp.zeros((16,), jnp.float32)
    pltpu.sync_copy(degv_ref, deg_hbm.at[pl.ds(base, _SC_ROWS_PER_W)])


def _sc_deg(D):
    return pl.kernel(
        _sc_deg_body,
        out_type=jax.ShapeDtypeStruct((_SC_ROWS,), jnp.float32),
        mesh=plsc.VectorSubcoreMesh(core_axis_name="c", subcore_axis_name="s"),
        scratch_types=[
            pltpu.VMEM((2, _SC_CHUNK, _N), jnp.float32),
            pltpu.VMEM((_SC_ROWS_PER_W,), jnp.float32),
            pltpu.SemaphoreType.DMA,
            pltpu.SemaphoreType.DMA,
        ],
    )(D)


def _acc_step(ab, af, xs, d, col0, acc_ref, gfirst_ref, gval_ref):
    acc_ref[...] += jnp.dot(af, xs, preferred_element_type=jnp.float32)
    iota = jax.lax.broadcasted_iota(jnp.int32, ab.shape, 1) + col0
    masked = jnp.where(ab, iota, _N)
    lmin = jnp.min(masked, axis=1, keepdims=True)
    onehot = (iota == lmin).astype(jnp.float32)
    lval = jnp.dot(onehot, d, preferred_element_type=jnp.float32)
    upd = lmin < gfirst_ref[...]
    gval_ref[...] = jnp.where(upd, lval, gval_ref[...])
    gfirst_ref[...] = jnp.where(upd, lmin, gfirst_ref[...])


def _tc1_body(d_ref, x_ref, a_ref, acc_ref, gfirst_ref, gval_ref):
    k = pl.program_id(0)

    @pl.when(k == 0)
    def _init():
        acc_ref[...] = jnp.zeros_like(acc_ref)
        gfirst_ref[...] = jnp.full_like(gfirst_ref, _N)
        gval_ref[...] = jnp.ones_like(gval_ref)

    d = jnp.max(d_ref[...], axis=1, keepdims=True)
    xs = (x_ref[...] * jax.lax.rsqrt(d)).astype(jnp.bfloat16)
    ab = a_ref[...] > 0
    af = ab.astype(jnp.bfloat16)
    _acc_step(ab, af, xs, d, (k + _SC_SLABS) * _BK,
              acc_ref, gfirst_ref, gval_ref)


def _tc2_body(deg_ref, x_ref, a_ref, acc1_ref, gf1_ref, gv1_ref, w_ref, b_ref,
              o_ref, gfirst_ref, gval_ref):
    k = pl.program_id(0)
    nsteps = pl.num_programs(0)

    @pl.when(k == 0)
    def _init():
        o_ref[...] = acc1_ref[...]
        gfirst_ref[...] = gf1_ref[...]
        gval_ref[...] = gv1_ref[...]

    d = deg_ref[...]
    xs = (x_ref[...] * jax.lax.rsqrt(d)).astype(jnp.bfloat16)
    ab = a_ref[...] > 0
    af = ab.astype(jnp.bfloat16)
    _acc_step(ab, af, xs, d, k * _BK, o_ref, gfirst_ref, gval_ref)

    @pl.when(k == nsteps - 1)
    def _epilogue():
        c = jax.lax.rsqrt(gval_ref[...])
        z = jax.lax.dot_general(
            o_ref[...], w_ref[...], (((1,), (1,)), ((), ())),
            preferred_element_type=jnp.float32)
        z = z * c + b_ref[...]
        o_ref[...] = jnp.where(z >= 0.0, z, 0.01 * z)


@jax.jit
def kernel(D, X, A, W, b):
    n, in_ch = X.shape
    out_ch = W.shape[0]

    acc1, gf1, gv1 = pl.pallas_call(
        _tc1_body,
        grid=(_NSLAB - _SC_SLABS,),
        in_specs=[
            pl.BlockSpec((_BK, n), lambda k: (k + _SC_SLABS, 0)),
            pl.BlockSpec((_BK, in_ch), lambda k: (k + _SC_SLABS, 0)),
            pl.BlockSpec((n, _BK), lambda k: (0, k + _SC_SLABS)),
        ],
        out_specs=[
            pl.BlockSpec((n, out_ch), lambda k: (0, 0)),
            pl.BlockSpec((n, 1), lambda k: (0, 0)),
            pl.BlockSpec((n, 1), lambda k: (0, 0)),
        ],
        out_shape=[
            jax.ShapeDtypeStruct((n, out_ch), jnp.float32),
            jax.ShapeDtypeStruct((n, 1), jnp.int32),
            jax.ShapeDtypeStruct((n, 1), jnp.float32),
        ],
    )(D, X, A)

    deg_sc = _sc_deg(D).reshape(_SC_ROWS, 1)

    out = pl.pallas_call(
        _tc2_body,
        grid=(_SC_SLABS,),
        in_specs=[
            pl.BlockSpec((_BK, 1), lambda k: (k, 0)),
            pl.BlockSpec((_BK, in_ch), lambda k: (k, 0)),
            pl.BlockSpec((n, _BK), lambda k: (0, k)),
            pl.BlockSpec((n, out_ch), lambda k: (0, 0)),
            pl.BlockSpec((n, 1), lambda k: (0, 0)),
            pl.BlockSpec((n, 1), lambda k: (0, 0)),
            pl.BlockSpec((out_ch, in_ch), lambda k: (0, 0)),
            pl.BlockSpec((1, out_ch), lambda k: (0, 0)),
        ],
        out_specs=pl.BlockSpec((n, out_ch), lambda k: (0, 0)),
        out_shape=jax.ShapeDtypeStruct((n, out_ch), jnp.float32),
        scratch_shapes=[
            pltpu.VMEM((n, 1), jnp.int32),
            pltpu.VMEM((n, 1), jnp.float32),
        ],
    )(deg_sc, X, A, acc1, gf1, gv1, W, b.reshape(1, out_ch))
    return out


# speedup vs baseline: 1.6190x; 1.6190x over previous
import jax
import jax.numpy as jnp
from jax.experimental import pallas as pl
from jax.experimental.pallas import tpu as pltpu

_N = 4096
_BK = 512


def _fused_body(d_ref, x_ref, a_ref, w_ref, b_ref, o_ref,
                acc_ref, gfirst_ref, gval_ref):
    k = pl.program_id(0)
    nsteps = pl.num_programs(0)

    @pl.when(k == 0)
    def _init():
        acc_ref[...] = jnp.zeros_like(acc_ref)
        gfirst_ref[...] = jnp.full_like(gfirst_ref, _N)
        gval_ref[...] = jnp.ones_like(gval_ref)

    d = jnp.max(d_ref[...], axis=1, keepdims=True)
    xs = (x_ref[...] * jax.lax.rsqrt(d)).astype(jnp.bfloat16)

    a = a_ref[...]
    ab = a > 0
    af = ab.astype(jnp.bfloat16)
    acc_ref[...] += jnp.dot(af, xs, preferred_element_type=jnp.float32)

    iota = jax.lax.broadcasted_iota(jnp.int32, a.shape, 1) + k * _BK
    masked = jnp.where(ab, iota, _N)
    lmin = jnp.min(masked, axis=1, keepdims=True)
    onehot = (iota == lmin).astype(jnp.float32)
    lval = jnp.dot(onehot, d, preferred_element_type=jnp.float32)
    upd = lmin < gfirst_ref[...]
    gval_ref[...] = jnp.where(upd, lval, gval_ref[...])
    gfirst_ref[...] = jnp.where(upd, lmin, gfirst_ref[...])

    @pl.when(k == nsteps - 1)
    def _epilogue():
        c = jax.lax.rsqrt(gval_ref[...])
        z = jax.lax.dot_general(
            acc_ref[...], w_ref[...], (((1,), (1,)), ((), ())),
            preferred_element_type=jnp.float32)
        z = z * c + b_ref[...]
        o_ref[...] = jnp.where(z >= 0.0, z, 0.01 * z)


@jax.jit
def kernel(D, X, A, W, b):
    n, in_ch = X.shape
    out_ch = W.shape[0]

    out = pl.pallas_call(
        _fused_body,
        grid=(n // _BK,),
        in_specs=[
            pl.BlockSpec((_BK, n), lambda k: (k, 0)),
            pl.BlockSpec((_BK, in_ch), lambda k: (k, 0)),
            pl.BlockSpec((n, _BK), lambda k: (0, k)),
            pl.BlockSpec((out_ch, in_ch), lambda k: (0, 0)),
            pl.BlockSpec((1, out_ch), lambda k: (0, 0)),
        ],
        out_specs=pl.BlockSpec((n, out_ch), lambda k: (0, 0)),
        out_shape=jax.ShapeDtypeStruct((n, out_ch), jnp.float32),
        scratch_shapes=[
            pltpu.VMEM((n, out_ch), jnp.float32),
            pltpu.VMEM((n, 1), jnp.int32),
            pltpu.VMEM((n, 1), jnp.float32),
        ],
    )(D, X, A, W, b.reshape(1, out_ch))
    return out
